# baseline (speedup 1.0000x reference)
import functools

import jax
import jax.numpy as jnp
from jax import lax
from jax.experimental import pallas as pl
from jax.experimental.pallas import tpu as pltpu
from jax.experimental.pallas import tpu_sc as plsc

N = 10000
E_RAW = 160000
E_TOT = E_RAW + N
NUM_GRAPHS = 16

C = 128
NT = 16
CH = -(-E_TOT // (NT * C))
E_PAD = NT * C * CH
N_ACC = 10240
ROWS_T = N_ACC // NT
AD_PAD = 10016

R = 2000
NB = N // R


def _make_edge_agg(half, edge_split):
    mesh = plsc.VectorSubcoreMesh(core_axis_name="c", subcore_axis_name="s")
    n_lastcopy = N - (NT - 1) * ROWS_T
    ch = CH // 2 if edge_split else CH

    def body(hs, asp, adp, srcp, dstp, out_hbm, den_hbm,
             srcv, dstv, srcv2, asg, adg, exc, rowbuf,
             srcv_b, dstv_b, srcv2_b, asg_b, adg_b, exc_b, rowbuf_b,
             denv, sem, sem_b, sem_s, sem_s_b, acc, den):
        c = lax.axis_index("c")
        s = lax.axis_index("s")
        z16 = jnp.zeros((16,), jnp.float32)
        bufs = ((srcv, dstv, srcv2, asg, adg, exc, rowbuf, sem, sem_s),
                (srcv_b, dstv_b, srcv2_b, asg_b, adg_b, exc_b, rowbuf_b,
                 sem_b, sem_s_b))

        def zrow(r, _):
            for j in range(half // 16):
                rowbuf[r, pl.ds(j * 16, 16)] = z16
            return 0
        lax.fori_loop(0, C, zrow, 0)
        for k in range(ROWS_T // C):
            pltpu.sync_copy(rowbuf, acc.at[pl.ds(s * ROWS_T + k * C, C)])
        for j in range(C // 16):
            asg[pl.ds(j * 16, 16)] = z16
        for k in range(ROWS_T // C):
            pltpu.sync_copy(asg, den.at[pl.ds(s * ROWS_T + k * C, C)])
        plsc.subcore_barrier()

        coff = c * N
        grp = (c * NT + s) if edge_split else s
        gbase = grp * ch * C

        def issue_idx(i, b):
            sv, dv, sv2, ag, dg, ex, rb, sm, ssm = bufs[b]
            pltpu.async_copy(srcp.at[pl.ds(gbase + i * C, C)], sv, sm)
            pltpu.async_copy(dstp.at[pl.ds(gbase + i * C, C)], dv, sm)

        def issue_gathers(i, b):
            sv, dv, sv2, ag, dg, ex, rb, sm, ssm = bufs[b]
            pltpu.make_async_copy(srcp.at[pl.ds(gbase + i * C, C)],
                                  sv, sm).wait()
            pltpu.make_async_copy(dstp.at[pl.ds(gbase + i * C, C)],
                                  dv, sm).wait()
            if edge_split:
                hidx = sv
            else:
                for j in range(C // 16):
                    sv2[pl.ds(j * 16, 16)] = sv[pl.ds(j * 16, 16)] + coff
                hidx = sv2
            pltpu.async_copy(asp.at[sv], ag, sm)
            pltpu.async_copy(adp.at[dv], dg, sm)
            pltpu.async_copy(hs.at[hidx], rb, sm)

        def process(b):
            sv, dv, sv2, ag, dg, ex, rb, sm, ssm = bufs[b]
            hidx = sv if edge_split else sv2
            pltpu.make_async_copy(asp.at[sv], ag, sm).wait()
            pltpu.make_async_copy(adp.at[dv], dg, sm).wait()
            pltpu.make_async_copy(hs.at[hidx], rb, sm).wait()
            for j in range(C // 16):
                e = ag[pl.ds(j * 16, 16)] + dg[pl.ds(j * 16, 16)]
                e = jnp.where(e >= 0.0, e, 0.2 * e)
                ex[pl.ds(j * 16, 16)] = jnp.exp(e)

            @plsc.parallel_loop(0, C, 1, unroll=4)
            def rowscale(r):
                idx = jnp.zeros((16,), jnp.int32) + r
                sp = plsc.load_gather(ex, [idx])
                for j2 in range(half // 16):
                    rb[r, pl.ds(j2 * 16, 16)] = rb[r, pl.ds(j2 * 16, 16)] * sp

            pltpu.async_copy(ex, den.at[dv], ssm, add=True)
            pltpu.async_copy(rb, acc.at[dv], ssm, add=True)

        def drain_scatters(b):
            sv, dv, sv2, ag, dg, ex, rb, sm, ssm = bufs[b]
            pltpu.make_async_copy(ex, den.at[dv], ssm).wait()
            pltpu.make_async_copy(rb, acc.at[dv], ssm).wait()

        np2 = ch // 2
        issue_idx(0, 0)
        issue_gathers(0, 0)
        issue_idx(1, 1)

        def pair(i2, _):
            i = 2 * i2

            @pl.when(i2 > 0)
            def _():
                drain_scatters(1)

            issue_gathers(i + 1, 1)
            process(0)

            @pl.when(i2 + 1 < np2)
            def _():
                issue_idx(i + 2, 0)

            process(1)

            @pl.when(i2 + 1 < np2)
            def _():
                drain_scatters(0)
                issue_gathers(i + 2, 0)
                issue_idx(i + 3, 1)

            return 0

        lax.fori_loop(0, np2, pair, 0)
        drain_scatters(0)
        drain_scatters(1)
        plsc.subcore_barrier()

        ro = coff + s * ROWS_T

        @pl.when(s < NT - 1)
        def _():
            pltpu.sync_copy(acc.at[pl.ds(s * ROWS_T, ROWS_T)],
                            out_hbm.at[pl.ds(ro, ROWS_T)])

        @pl.when(s == NT - 1)
        def _():
            pltpu.sync_copy(acc.at[pl.ds(s * ROWS_T, n_lastcopy)],
                            out_hbm.at[pl.ds(ro, n_lastcopy)])

        den_write = (s < 10) if edge_split else jnp.logical_and(c == 0, s < 10)
        doff = coff + s * 1000 if edge_split else s * 1000

        @pl.when(den_write)
        def _():
            pltpu.sync_copy(den.at[pl.ds(s * 1000, 1000)], denv)
            pltpu.sync_copy(denv, den_hbm.at[pl.ds(doff, 1000)])

    return pl.kernel(
        body,
        mesh=mesh,
        compiler_params=pltpu.CompilerParams(needs_layout_passes=False),
        out_type=[jax.ShapeDtypeStruct((2 * N, half), jnp.float32),
                  jax.ShapeDtypeStruct(((2 * N if edge_split else N),),
                                       jnp.float32)],
        scratch_types=(
            ([pltpu.VMEM((C,), jnp.int32)] * 3
               + [pltpu.VMEM((C,), jnp.float32)] * 3
               + [pltpu.VMEM((C, half), jnp.float32)]) * 2
            + [pltpu.VMEM((1000,), jnp.float32),
               pltpu.SemaphoreType.DMA,
               pltpu.SemaphoreType.DMA,
               pltpu.SemaphoreType.DMA,
               pltpu.SemaphoreType.DMA,
               pltpu.VMEM_SHARED((N_ACC, half), jnp.float32),
               pltpu.VMEM_SHARED((N_ACC,), jnp.float32)]
        ),
    )


_make_edge_agg = functools.lru_cache(maxsize=None)(_make_edge_agg)


def _edge_agg(hs, asp, adp, srcp, dstp, half, edge_split=False):
    return _make_edge_agg(half, edge_split)(hs, asp, adp, srcp, dstp)


def _tc_first(x, wall, hout, half):
    hcat = hout + 128

    def body(x_ref, w_ref, h0_ref, h1_ref, asd_ref):
        h = jnp.dot(x_ref[...], w_ref[...], preferred_element_type=jnp.float32)
        h0_ref[...] = h[:, :half]
        h1_ref[...] = h[:, half:2 * half]
        asd_ref[...] = h[:, hout:hout + 128]

    return pl.pallas_call(
        body,
        grid=(NB,),
        in_specs=[pl.BlockSpec((R, x.shape[1]), lambda i: (i, 0)),
                  pl.BlockSpec((x.shape[1], hcat), lambda i: (0, 0))],
        out_specs=[pl.BlockSpec((R, half), lambda i: (i, 0)),
                   pl.BlockSpec((R, half), lambda i: (i, 0)),
                   pl.BlockSpec((R, 128), lambda i: (i, 0))],
        out_shape=[jax.ShapeDtypeStruct((N, half), jnp.float32),
                   jax.ShapeDtypeStruct((N, half), jnp.float32),
                   jax.ShapeDtypeStruct((N, 128), jnp.float32)],
    )(x, wall)


def _tc_mid(parts, den, b, wall, hin_half, hout, half):
    din = 2 * hin_half
    hcat = hout + 128
    split = half is not None

    def body(x_ref, d_ref, b_ref, w_ref, *out_refs):
        xcat = jnp.concatenate([x_ref[0], x_ref[1]], axis=1)
        pre = jnp.maximum(xcat / d_ref[...] + b_ref[...], 0.0)
        h = jnp.dot(pre, w_ref[...], preferred_element_type=jnp.float32)
        if split:
            h0_ref, h1_ref, asd_ref = out_refs
            h0_ref[...] = h[:, :half]
            h1_ref[...] = h[:, half:2 * half]
        else:
            h0_ref, asd_ref = out_refs
            h0_ref[...] = h[:, :hout]
        asd_ref[...] = h[:, hout:hout + 128]

    if split:
        houts = [pl.BlockSpec((R, half), lambda i: (i, 0))] * 2
        hshapes = [jax.ShapeDtypeStruct((N, half), jnp.float32)] * 2
    else:
        houts = [pl.BlockSpec((R, hout), lambda i: (i, 0))]
        hshapes = [jax.ShapeDtypeStruct((N, hout), jnp.float32)]
    return pl.pallas_call(
        body,
        grid=(NB,),
        in_specs=[pl.BlockSpec((2, R, hin_half), lambda i: (0, i, 0)),
                  pl.BlockSpec((R, 1), lambda i: (i, 0)),
                  pl.BlockSpec((1, din), lambda i: (0, 0)),
                  pl.BlockSpec((din, hcat), lambda i: (0, 0))],
        out_specs=houts + [pl.BlockSpec((R, 128), lambda i: (i, 0))],
        out_shape=hshapes + [jax.ShapeDtypeStruct((N, 128), jnp.float32)],
    )(parts, den, b, wall)


def _tc_pool(parts, den, b, ids):
    def body(x_ref, d_ref, b_ref, ids_ref, o_ref, acc_s, acc_c):
        i = pl.program_id(0)

        @pl.when(i == 0)
        def _():
            acc_s[...] = jnp.zeros_like(acc_s)
            acc_c[...] = jnp.zeros_like(acc_c)

        xsum = x_ref[0] + x_ref[1]
        dsum = d_ref[0] + d_ref[1]
        pre = jnp.maximum(xsum / dsum + b_ref[...], 0.0)
        g = lax.broadcasted_iota(jnp.int32, (NUM_GRAPHS, R), 0)
        oh = (g == ids_ref[0]).astype(jnp.float32)
        acc_s[...] += jnp.dot(oh, pre, preferred_element_type=jnp.float32)
        cnt = jnp.sum(oh, axis=1)
        acc_c[...] += jnp.broadcast_to(cnt[:, None], (NUM_GRAPHS, 128))

        @pl.when(i == NB - 1)
        def _():
            o_ref[...] = acc_s[...] / jnp.maximum(acc_c[...], 1.0)

    return pl.pallas_call(
        body,
        grid=(NB,),
        in_specs=[pl.BlockSpec((2, R, 128), lambda i: (0, i, 0)),
                  pl.BlockSpec((2, R, 1), lambda i: (0, i, 0)),
                  pl.BlockSpec((1, 128), lambda i: (0, 0)),
                  pl.BlockSpec((1, 1, R), lambda i: (i, 0, 0))],
        out_specs=pl.BlockSpec((NUM_GRAPHS, 128), lambda i: (0, 0)),
        out_shape=jax.ShapeDtypeStruct((NUM_GRAPHS, 128), jnp.float32),
        scratch_shapes=[pltpu.VMEM((NUM_GRAPHS, 128), jnp.float32),
                        pltpu.VMEM((NUM_GRAPHS, 128), jnp.float32)],
    )(parts, den, b, ids)


def _wall(W, a_s, a_d):
    wasd = jnp.pad(jnp.stack([W @ a_s, W @ a_d], axis=1), ((0, 0), (0, 126)))
    return jnp.concatenate([W, wasd], axis=1)


def kernel(x, edge_index, batch, W1, a1s, a1d, b1, W2, a2s, a2d, b2,
           W3, a3s, a3d, b3):
    loops = jnp.arange(N, dtype=jnp.int32)
    src = jnp.concatenate([edge_index[0], loops])
    dst = jnp.concatenate([edge_index[1], loops])
    order = jnp.argsort(src)
    src = src[order]
    dst = dst[order]
    padn = E_PAD - E_TOT
    srcp = jnp.concatenate([src, jnp.zeros((padn,), jnp.int32)])
    dstp = jnp.concatenate([dst, jnp.full((padn,), N, jnp.int32)])

    h0, h1, asd = _tc_first(x, _wall(W1, a1s, a1d), 256, 128)
    hs = jnp.concatenate([h0, h1], axis=0)
    agg1, den1 = _edge_agg(hs, asd[:, 0], jnp.pad(asd[:, 1], (0, AD_PAD - N)),
                           srcp, dstp, 128)

    h0, h1, asd = _tc_mid(agg1.reshape(2, N, 128), den1[:, None], b1[None, :],
                          _wall(W2, a2s, a2d), 128, 256, 128)
    hs = jnp.concatenate([h0, h1], axis=0)
    agg2, den2 = _edge_agg(hs, asd[:, 0], jnp.pad(asd[:, 1], (0, AD_PAD - N)),
                           srcp, dstp, 128)

    h3, asd = _tc_mid(agg2.reshape(2, N, 128), den2[:, None], b2[None, :],
                      _wall(W3, a3s, a3d), 128, 128, None)
    agg3, den3 = _edge_agg(h3, asd[:, 0], jnp.pad(asd[:, 1], (0, AD_PAD - N)),
                           srcp, dstp, 128, edge_split=True)

    ids = batch.astype(jnp.int32).reshape(NB, 1, R)
    return _tc_pool(agg3.reshape(2, N, 128), den3.reshape(2, N, 1),
                    b3[None, :], ids)


# speedup vs baseline: 1.3725x; 1.3725x over previous
import functools

import numpy as np

import jax
import jax.numpy as jnp
from jax import lax
from jax.experimental import pallas as pl
from jax.experimental.pallas import tpu as pltpu
from jax.experimental.pallas import tpu_sc as plsc

N = 10000
E_RAW = 160000
E_TOT = E_RAW + N
NUM_GRAPHS = 16

C = 128
NT = 16
CH = 2 * (-(-E_TOT // (NT * C * 2)))
E_PAD = NT * C * CH
N_ACC = 10240
ROWS_T = N_ACC // NT
AD_PAD = 10016

R = 2000
NB = N // R


def _make_edge_agg(half, edge_split):
    mesh = plsc.VectorSubcoreMesh(core_axis_name="c", subcore_axis_name="s")
    n_lastcopy = N - (NT - 1) * ROWS_T
    ch = CH // 2 if edge_split else CH

    def body(hs, asp, adp, srcp, dstp, out_hbm, den_hbm,
             srcv, dstv, srcv2, asg, adg, exc, rowbuf,
             srcv_b, dstv_b, srcv2_b, asg_b, adg_b, exc_b, rowbuf_b,
             denv, sem, sem_b, sem_s, sem_s_b, acc, den):
        c = lax.axis_index("c")
        s = lax.axis_index("s")
        z16 = jnp.zeros((16,), jnp.float32)
        bufs = ((srcv, dstv, srcv2, asg, adg, exc, rowbuf, sem, sem_s),
                (srcv_b, dstv_b, srcv2_b, asg_b, adg_b, exc_b, rowbuf_b,
                 sem_b, sem_s_b))

        def zrow(r, _):
            for j in range(half // 16):
                rowbuf[r, pl.ds(j * 16, 16)] = z16
            return 0
        lax.fori_loop(0, C, zrow, 0)
        zfull, zrem = divmod(ROWS_T, C)
        for k in range(zfull):
            pltpu.sync_copy(rowbuf, acc.at[pl.ds(s * ROWS_T + k * C, C)])
        if zrem:
            pltpu.sync_copy(rowbuf.at[pl.ds(0, zrem)],
                            acc.at[pl.ds(s * ROWS_T + zfull * C, zrem)])
        for j in range(C // 16):
            asg[pl.ds(j * 16, 16)] = z16
        for k in range(zfull):
            pltpu.sync_copy(asg, den.at[pl.ds(s * ROWS_T + k * C, C)])
        if zrem:
            pltpu.sync_copy(asg.at[pl.ds(0, zrem)],
                            den.at[pl.ds(s * ROWS_T + zfull * C, zrem)])
        plsc.subcore_barrier()

        coff = c * N
        grp = (c * NT + s) if edge_split else s
        gbase = grp * ch * C

        def issue_idx(i, b):
            sv, dv, sv2, ag, dg, ex, rb, sm, ssm = bufs[b]
            pltpu.async_copy(srcp.at[pl.ds(gbase + i * C, C)], sv, sm)
            pltpu.async_copy(dstp.at[pl.ds(gbase + i * C, C)], dv, sm)

        def issue_gathers(i, b):
            sv, dv, sv2, ag, dg, ex, rb, sm, ssm = bufs[b]
            pltpu.make_async_copy(srcp.at[pl.ds(gbase + i * C, C)],
                                  sv, sm).wait()
            pltpu.make_async_copy(dstp.at[pl.ds(gbase + i * C, C)],
                                  dv, sm).wait()
            if edge_split:
                hidx = sv
            else:
                for j in range(C // 16):
                    sv2[pl.ds(j * 16, 16)] = sv[pl.ds(j * 16, 16)] + coff
                hidx = sv2
            pltpu.async_copy(asp.at[sv], ag, sm)
            pltpu.async_copy(adp.at[dv], dg, sm)
            pltpu.async_copy(hs.at[hidx], rb, sm)

        def process(b):
            sv, dv, sv2, ag, dg, ex, rb, sm, ssm = bufs[b]
            hidx = sv if edge_split else sv2
            pltpu.make_async_copy(asp.at[sv], ag, sm).wait()
            pltpu.make_async_copy(adp.at[dv], dg, sm).wait()
            pltpu.make_async_copy(hs.at[hidx], rb, sm).wait()
            for j in range(C // 16):
                e = ag[pl.ds(j * 16, 16)] + dg[pl.ds(j * 16, 16)]
                e = jnp.where(e >= 0.0, e, 0.2 * e)
                ex[pl.ds(j * 16, 16)] = jnp.exp(e)

            @plsc.parallel_loop(0, C, 1, unroll=4)
            def rowscale(r):
                idx = jnp.zeros((16,), jnp.int32) + r
                sp = plsc.load_gather(ex, [idx])
                for j2 in range(half // 16):
                    rb[r, pl.ds(j2 * 16, 16)] = rb[r, pl.ds(j2 * 16, 16)] * sp

            pltpu.async_copy(ex, den.at[dv], ssm, add=True)
            pltpu.async_copy(rb, acc.at[dv], ssm, add=True)

        def drain_scatters(b):
            sv, dv, sv2, ag, dg, ex, rb, sm, ssm = bufs[b]
            pltpu.make_async_copy(ex, den.at[dv], ssm).wait()
            pltpu.make_async_copy(rb, acc.at[dv], ssm).wait()

        np2 = ch // 2
        issue_idx(0, 0)
        issue_gathers(0, 0)
        issue_idx(1, 1)

        def pair(i2, _):
            i = 2 * i2

            @pl.when(i2 > 0)
            def _():
                drain_scatters(1)

            issue_gathers(i + 1, 1)
            process(0)

            @pl.when(i2 + 1 < np2)
            def _():
                issue_idx(i + 2, 0)

            process(1)

            @pl.when(i2 + 1 < np2)
            def _():
                drain_scatters(0)
                issue_gathers(i + 2, 0)
                issue_idx(i + 3, 1)

            return 0

        lax.fori_loop(0, np2, pair, 0)
        drain_scatters(0)
        drain_scatters(1)
        plsc.subcore_barrier()

        ro = coff + s * ROWS_T

        @pl.when(s < NT - 1)
        def _():
            pltpu.sync_copy(acc.at[pl.ds(s * ROWS_T, ROWS_T)],
                            out_hbm.at[pl.ds(ro, ROWS_T)])

        @pl.when(s == NT - 1)
        def _():
            pltpu.sync_copy(acc.at[pl.ds(s * ROWS_T, n_lastcopy)],
                            out_hbm.at[pl.ds(ro, n_lastcopy)])

        den_write = (s < 10) if edge_split else jnp.logical_and(c == 0, s < 10)
        doff = coff + s * 1000 if edge_split else s * 1000

        @pl.when(den_write)
        def _():
            pltpu.sync_copy(den.at[pl.ds(s * 1000, 1000)], denv)
            pltpu.sync_copy(denv, den_hbm.at[pl.ds(doff, 1000)])

    return pl.kernel(
        body,
        mesh=mesh,
        compiler_params=pltpu.CompilerParams(needs_layout_passes=False),
        out_type=[jax.ShapeDtypeStruct((2 * N, half), jnp.float32),
                  jax.ShapeDtypeStruct(((2 * N if edge_split else N),),
                                       jnp.float32)],
        scratch_types=(
            ([pltpu.VMEM((C,), jnp.int32)] * 3
               + [pltpu.VMEM((C,), jnp.float32)] * 3
               + [pltpu.VMEM((C, half), jnp.float32)]) * 2
            + [pltpu.VMEM((1000,), jnp.float32),
               pltpu.SemaphoreType.DMA,
               pltpu.SemaphoreType.DMA,
               pltpu.SemaphoreType.DMA,
               pltpu.SemaphoreType.DMA,
               pltpu.VMEM_SHARED((N_ACC, half), jnp.float32),
               pltpu.VMEM_SHARED((N_ACC,), jnp.float32)]
        ),
    )


_make_edge_agg = functools.lru_cache(maxsize=None)(_make_edge_agg)


def _edge_agg(hs, asp, adp, srcp, dstp, half, edge_split=False):
    return _make_edge_agg(half, edge_split)(hs, asp, adp, srcp, dstp)


def _tc_first(x, wall, hout, half):
    hcat = hout + 128

    def body(x_ref, w_ref, h0_ref, h1_ref, asd_ref):
        h = jnp.dot(x_ref[...], w_ref[...], preferred_element_type=jnp.float32)
        h0_ref[...] = h[:, :half]
        h1_ref[...] = h[:, half:2 * half]
        asd_ref[...] = h[:, hout:hout + 128]

    return pl.pallas_call(
        body,
        grid=(NB,),
        in_specs=[pl.BlockSpec((R, x.shape[1]), lambda i: (i, 0)),
                  pl.BlockSpec((x.shape[1], hcat), lambda i: (0, 0))],
        out_specs=[pl.BlockSpec((R, half), lambda i: (i, 0)),
                   pl.BlockSpec((R, half), lambda i: (i, 0)),
                   pl.BlockSpec((R, 128), lambda i: (i, 0))],
        out_shape=[jax.ShapeDtypeStruct((N, half), jnp.float32),
                   jax.ShapeDtypeStruct((N, half), jnp.float32),
                   jax.ShapeDtypeStruct((N, 128), jnp.float32)],
    )(x, wall)


def _tc_mid(parts, den, b, wall, hin_half, hout, half):
    din = 2 * hin_half
    hcat = hout + 128
    split = half is not None

    def body(x_ref, d_ref, b_ref, w_ref, *out_refs):
        xcat = jnp.concatenate([x_ref[0], x_ref[1]], axis=1)
        pre = jnp.maximum(xcat / d_ref[...] + b_ref[...], 0.0)
        h = jnp.dot(pre, w_ref[...], preferred_element_type=jnp.float32)
        if split:
            h0_ref, h1_ref, asd_ref = out_refs
            h0_ref[...] = h[:, :half]
            h1_ref[...] = h[:, half:2 * half]
        else:
            h0_ref, asd_ref = out_refs
            h0_ref[...] = h[:, :hout]
        asd_ref[...] = h[:, hout:hout + 128]

    if split:
        houts = [pl.BlockSpec((R, half), lambda i: (i, 0))] * 2
        hshapes = [jax.ShapeDtypeStruct((N, half), jnp.float32)] * 2
    else:
        houts = [pl.BlockSpec((R, hout), lambda i: (i, 0))]
        hshapes = [jax.ShapeDtypeStruct((N, hout), jnp.float32)]
    return pl.pallas_call(
        body,
        grid=(NB,),
        in_specs=[pl.BlockSpec((2, R, hin_half), lambda i: (0, i, 0)),
                  pl.BlockSpec((R, 1), lambda i: (i, 0)),
                  pl.BlockSpec((1, din), lambda i: (0, 0)),
                  pl.BlockSpec((din, hcat), lambda i: (0, 0))],
        out_specs=houts + [pl.BlockSpec((R, 128), lambda i: (i, 0))],
        out_shape=hshapes + [jax.ShapeDtypeStruct((N, 128), jnp.float32)],
    )(parts, den, b, wall)


def _tc_pool(parts, den, b, ids, m):
    def body(x_ref, d_ref, b_ref, ids_ref, m_ref, o_ref, acc_s, acc_c):
        i = pl.program_id(0)

        @pl.when(i == 0)
        def _():
            acc_s[...] = jnp.zeros_like(acc_s)
            acc_c[...] = jnp.zeros_like(acc_c)

        xsum = x_ref[0] + x_ref[1]
        dsum = d_ref[0] + d_ref[1]
        pre = jnp.maximum(xsum / dsum + b_ref[...], 0.0)
        g = lax.broadcasted_iota(jnp.int32, (NUM_GRAPHS, R), 0)
        oh = (g == ids_ref[0]).astype(jnp.float32)
        acc_s[...] += jnp.dot(oh, pre, preferred_element_type=jnp.float32)
        cnt = jnp.sum(oh, axis=1)
        acc_c[...] += jnp.broadcast_to(cnt[:, None], (NUM_GRAPHS, 128))

        @pl.when(i == NB - 1)
        def _():
            o_ref[...] = jnp.dot(acc_s[...] / jnp.maximum(acc_c[...], 1.0),
                                 m_ref[...], preferred_element_type=jnp.float32)

    return pl.pallas_call(
        body,
        grid=(NB,),
        in_specs=[pl.BlockSpec((2, R, 128), lambda i: (0, i, 0)),
                  pl.BlockSpec((2, R, 1), lambda i: (0, i, 0)),
                  pl.BlockSpec((1, 128), lambda i: (0, 0)),
                  pl.BlockSpec((1, 1, R), lambda i: (i, 0, 0)),
                  pl.BlockSpec((128, 128), lambda i: (0, 0))],
        out_specs=pl.BlockSpec((NUM_GRAPHS, 128), lambda i: (0, 0)),
        out_shape=jax.ShapeDtypeStruct((NUM_GRAPHS, 128), jnp.float32),
        scratch_shapes=[pltpu.VMEM((NUM_GRAPHS, 128), jnp.float32),
                        pltpu.VMEM((NUM_GRAPHS, 128), jnp.float32)],
    )(parts, den, b, ids, m)


_EYE128 = np.eye(128, dtype=np.float32)


def _wall(W, a_s, a_d):
    wasd = jnp.pad(jnp.stack([W @ a_s, W @ a_d], axis=1), ((0, 0), (0, 126)))
    return jnp.concatenate([W, wasd], axis=1)


def kernel(x, edge_index, batch, W1, a1s, a1d, b1, W2, a2s, a2d, b2,
           W3, a3s, a3d, b3):
    loops = jnp.arange(N, dtype=jnp.int32)
    src = jnp.concatenate([edge_index[0], loops])
    dst = jnp.concatenate([edge_index[1], loops])
    padn = E_PAD - E_TOT
    srcp = jnp.concatenate([src, jnp.zeros((padn,), jnp.int32)])
    dstp = jnp.concatenate([dst, jnp.full((padn,), N, jnp.int32)])

    h0, h1, asd = _tc_first(x, _wall(W1, a1s, a1d), 256, 128)
    hs = jnp.concatenate([h0, h1], axis=0)
    agg1, den1 = _edge_agg(hs, asd[:, 0], jnp.pad(asd[:, 1], (0, AD_PAD - N)),
                           srcp, dstp, 128)

    h0, h1, asd = _tc_mid(agg1.reshape(2, N, 128), den1[:, None],
                          b1[None, :], _wall(W2, a2s, a2d), 128, 256, 128)
    hs = jnp.concatenate([h0, h1], axis=0)
    agg2, den2 = _edge_agg(hs, asd[:, 0], jnp.pad(asd[:, 1], (0, AD_PAD - N)),
                           srcp, dstp, 128)

    h3, asd = _tc_mid(agg2.reshape(2, N, 128), den2[:, None],
                      b2[None, :], _wall(W3, a3s, a3d), 128, 128, None)
    agg3, den3 = _edge_agg(h3, asd[:, 0], jnp.pad(asd[:, 1], (0, AD_PAD - N)),
                           srcp, dstp, 128, edge_split=True)

    ids = batch.astype(jnp.int32).reshape(NB, 1, R)
    return _tc_pool(agg3.reshape(2, N, 128), den3.reshape(2, N, 1),
                    b3[None, :], ids, jnp.asarray(_EYE128))
